# baseline (speedup 1.0000x reference)
import functools

import jax
import jax.numpy as jnp
from jax import lax
from jax.experimental import pallas as pl
from jax.experimental.pallas import tpu as pltpu
from jax.experimental.pallas import tpu_sc as plsc

_B, _L, _D = 4096, 50, 64
_T = _B * _L
_NC, _NS = 2, 16
_NW = _NC * _NS
_TPW = _T // _NW
_CHUNK = 40
_NCHUNK = _TPW // _CHUNK
_M5 = 5 * _CHUNK


def _sc_gather_pool(item_f, genre_f, dir_f, wri_f,
                    item_table, genre_table, director_table, writer_table):
    mesh = plsc.VectorSubcoreMesh(core_axis_name="c", subcore_axis_name="s")

    @functools.partial(
        pl.kernel,
        out_type=(
            jax.ShapeDtypeStruct((_T, 128), jnp.float32),
            jax.ShapeDtypeStruct((_T, 128), jnp.float32),
        ),
        mesh=mesh,
        scratch_types=(
            pltpu.VMEM((2, _CHUNK), jnp.int32),
            pltpu.VMEM((2, 3, _M5), jnp.int32),
            pltpu.VMEM((2, _CHUNK, _D), jnp.float32),
            pltpu.VMEM((2, 3, _M5, _D), jnp.float32),
            pltpu.VMEM((2, _CHUNK, 128), jnp.float32),
            pltpu.VMEM((2, _CHUNK, 128), jnp.float32),
            pltpu.SemaphoreType.DMA,
            pltpu.SemaphoreType.DMA,
            pltpu.SemaphoreType.DMA,
            pltpu.SemaphoreType.DMA,
        ),
        compiler_params=pltpu.CompilerParams(use_tc_tiling_on_sc=False),
    )
    def k(if_hbm, gf_hbm, df_hbm, wf_hbm,
          item_hbm, genre_hbm, dir_hbm, wri_hbm,
          oa_hbm, ob_hbm, ii_v, ci_v, item_v, rows_v, oa_v, ob_v,
          gsem0, gsem1, isem0, isem1):
        wid = lax.axis_index("c") * _NS + lax.axis_index("s")
        gsems = (gsem0, gsem1)
        isems = (isem0, isem1)
        ctabs = (genre_hbm, dir_hbm, wri_hbm)
        cflats = (gf_hbm, df_hbm, wf_hbm)

        def idx_copies(b, kc, issue):
            t0 = wid * _TPW + kc * _CHUNK
            op = pltpu.async_copy if issue else pltpu.make_async_copy
            cps = [op(if_hbm.at[pl.ds(t0, _CHUNK)], ii_v.at[b], isems[b])]
            for t in range(3):
                cps.append(op(cflats[t].at[pl.ds(t0 * 5, _M5)], ci_v.at[b, t], isems[b]))
            return cps

        def gathers(b, issue):
            op = pltpu.async_copy if issue else pltpu.make_async_copy
            cps = [op(item_hbm.at[ii_v.at[b]], item_v.at[b], gsems[b])]
            for t in range(3):
                cps.append(op(ctabs[t].at[ci_v.at[b, t, pl.ds(0, 128)]],
                              rows_v.at[b, t, pl.ds(0, 128)], gsems[b]))
                cps.append(op(ctabs[t].at[ci_v.at[b, t, pl.ds(128, _M5 - 128)]],
                              rows_v.at[b, t, pl.ds(128, _M5 - 128)], gsems[b]))
            return cps

        for cp in idx_copies(0, 0, True):
            cp.wait()
        gathers(0, True)
        idx_copies(1, 1, True)

        @pl.loop(0, _NCHUNK, step=2)
        def _pair(k2):
            for b in range(2):
                kc = k2 + b
                nb = 1 - b

                @pl.when(kc + 1 < _NCHUNK)
                def _():
                    for cp in idx_copies(nb, kc + 1, False):
                        cp.wait()
                    gathers(nb, True)

                for cp in gathers(b, False):
                    cp.wait()

                @pl.when(kc + 2 < _NCHUNK)
                def _():
                    idx_copies(b, kc + 2, True)

                @pl.loop(0, _CHUNK, unroll=4)
                def _tok(i):
                    for c in range(_D // 16):
                        sl = pl.ds(c * 16, 16)
                        oa_v[b, i, sl] = item_v[b, i, sl]
                        for t in range(3):
                            acc = rows_v[b, t, 5 * i, sl]
                            for m in range(1, 5):
                                acc = acc + rows_v[b, t, 5 * i + m, sl]
                            if t == 0:
                                oa_v[b, i, pl.ds(_D + c * 16, 16)] = acc
                            else:
                                ob_v[b, i, pl.ds((t - 1) * _D + c * 16, 16)] = acc

                base = wid * _TPW + kc * _CHUNK
                pltpu.sync_copy(oa_v.at[b], oa_hbm.at[pl.ds(base, _CHUNK)])
                pltpu.sync_copy(ob_v.at[b], ob_hbm.at[pl.ds(base, _CHUNK)])

    return k(item_f, genre_f, dir_f, wri_f,
             item_table, genre_table, director_table, writer_table)


_BB = 64


def _tc_project(out_a, out_b, W, b):
    scale = jnp.float32(0.2)
    wa = jnp.concatenate([W[:_D], W[_D:2 * _D] * scale], axis=0)
    wb = jnp.concatenate([W[2 * _D:3 * _D] * scale, W[3 * _D:] * scale], axis=0)
    b2 = b.reshape(1, _D)

    def mm(ar, br_, war, wbr, biasr, o):
        x = (
            jnp.dot(ar[...], war[...], preferred_element_type=jnp.float32)
            + jnp.dot(br_[...], wbr[...], preferred_element_type=jnp.float32)
            + biasr[...]
        )
        o[...] = x.reshape(_BB, _L, _D)

    return pl.pallas_call(
        mm,
        grid=(_B // _BB,),
        in_specs=[
            pl.BlockSpec((_BB * _L, 128), lambda i: (i, 0)),
            pl.BlockSpec((_BB * _L, 128), lambda i: (i, 0)),
            pl.BlockSpec((128, _D), lambda i: (0, 0)),
            pl.BlockSpec((128, _D), lambda i: (0, 0)),
            pl.BlockSpec((1, _D), lambda i: (0, 0)),
        ],
        out_specs=pl.BlockSpec((_BB, _L, _D), lambda i: (i, 0, 0)),
        out_shape=jax.ShapeDtypeStruct((_B, _L, _D), jnp.float32),
    )(out_a, out_b, wa, wb, b2)


def kernel(item_ids, genre_ids, director_ids, writer_ids, item_table,
           genre_table, director_table, writer_table, W, b):
    item_f = item_ids.reshape(_T).astype(jnp.int32)
    genre_f = genre_ids.reshape(_T * 5).astype(jnp.int32)
    dir_f = director_ids.reshape(_T * 5).astype(jnp.int32)
    wri_f = writer_ids.reshape(_T * 5).astype(jnp.int32)

    out_a, out_b = _sc_gather_pool(
        item_f, genre_f, dir_f, wri_f,
        item_table, genre_table, director_table, writer_table
    )
    return _tc_project(out_a, out_b, W, b)


# speedup vs baseline: 1.0003x; 1.0003x over previous
import functools

import jax
import jax.numpy as jnp
from jax import lax
from jax.experimental import pallas as pl
from jax.experimental.pallas import tpu as pltpu
from jax.experimental.pallas import tpu_sc as plsc

_B, _L, _D = 4096, 50, 64
_T = _B * _L
_NC, _NS = 2, 16
_NW = _NC * _NS
_TPW = _T // _NW
_CHUNK = 40
_NCHUNK = _TPW // _CHUNK
_M5 = 5 * _CHUNK


def _sc_gather_pool(item_f, genre_f, dir_f, wri_f,
                    item_table, genre_table, director_table, writer_table):
    mesh = plsc.VectorSubcoreMesh(core_axis_name="c", subcore_axis_name="s")

    @functools.partial(
        pl.kernel,
        out_type=(
            jax.ShapeDtypeStruct((_T, 128), jnp.float32),
            jax.ShapeDtypeStruct((_T, 128), jnp.float32),
        ),
        mesh=mesh,
        scratch_types=(
            pltpu.VMEM((2, _CHUNK), jnp.int32),
            pltpu.VMEM((2, 3, _M5), jnp.int32),
            pltpu.VMEM((2, _CHUNK, _D), jnp.float32),
            pltpu.VMEM((2, 3, _M5, _D), jnp.float32),
            pltpu.VMEM((2, _CHUNK, 128), jnp.float32),
            pltpu.VMEM((2, _CHUNK, 128), jnp.float32),
            pltpu.SemaphoreType.DMA,
            pltpu.SemaphoreType.DMA,
            pltpu.SemaphoreType.DMA,
            pltpu.SemaphoreType.DMA,
        ),
        compiler_params=pltpu.CompilerParams(use_tc_tiling_on_sc=False),
    )
    def k(if_hbm, gf_hbm, df_hbm, wf_hbm,
          item_hbm, genre_hbm, dir_hbm, wri_hbm,
          oa_hbm, ob_hbm, ii_v, ci_v, item_v, rows_v, oa_v, ob_v,
          gsem0, gsem1, isem0, isem1):
        wid = lax.axis_index("c") * _NS + lax.axis_index("s")
        gsems = (gsem0, gsem1)
        isems = (isem0, isem1)
        ctabs = (genre_hbm, dir_hbm, wri_hbm)
        cflats = (gf_hbm, df_hbm, wf_hbm)

        def idx_copies(b, kc, issue):
            row = wid * _NCHUNK + kc
            op = pltpu.async_copy if issue else pltpu.make_async_copy
            cps = [op(if_hbm.at[row], ii_v.at[b], isems[b])]
            for t in range(3):
                cps.append(op(cflats[t].at[row], ci_v.at[b, t], isems[b]))
            return cps

        def gathers(b, issue):
            op = pltpu.async_copy if issue else pltpu.make_async_copy
            cps = [op(item_hbm.at[ii_v.at[b]], item_v.at[b], gsems[b])]
            for t in range(3):
                cps.append(op(ctabs[t].at[ci_v.at[b, t, pl.ds(0, 128)]],
                              rows_v.at[b, t, pl.ds(0, 128)], gsems[b]))
                cps.append(op(ctabs[t].at[ci_v.at[b, t, pl.ds(128, _M5 - 128)]],
                              rows_v.at[b, t, pl.ds(128, _M5 - 128)], gsems[b]))
            return cps

        for cp in idx_copies(0, 0, True):
            cp.wait()
        gathers(0, True)
        idx_copies(1, 1, True)

        @pl.loop(0, _NCHUNK, step=2)
        def _pair(k2):
            for b in range(2):
                kc = k2 + b
                nb = 1 - b

                @pl.when(kc + 1 < _NCHUNK)
                def _():
                    for cp in idx_copies(nb, kc + 1, False):
                        cp.wait()
                    gathers(nb, True)

                for cp in gathers(b, False):
                    cp.wait()

                @pl.when(kc + 2 < _NCHUNK)
                def _():
                    idx_copies(b, kc + 2, True)

                @pl.loop(0, _CHUNK, unroll=4)
                def _tok(i):
                    for c in range(_D // 16):
                        sl = pl.ds(c * 16, 16)
                        oa_v[b, i, sl] = item_v[b, i, sl]
                        for t in range(3):
                            acc = rows_v[b, t, 5 * i, sl]
                            for m in range(1, 5):
                                acc = acc + rows_v[b, t, 5 * i + m, sl]
                            if t == 0:
                                oa_v[b, i, pl.ds(_D + c * 16, 16)] = acc
                            else:
                                ob_v[b, i, pl.ds((t - 1) * _D + c * 16, 16)] = acc

                base = wid * _TPW + kc * _CHUNK
                pltpu.sync_copy(oa_v.at[b], oa_hbm.at[pl.ds(base, _CHUNK)])
                pltpu.sync_copy(ob_v.at[b], ob_hbm.at[pl.ds(base, _CHUNK)])

    return k(item_f, genre_f, dir_f, wri_f,
             item_table, genre_table, director_table, writer_table)


_BB = 64


def _tc_project(out_a, out_b, W, b):
    scale = jnp.float32(0.2)
    wa = jnp.concatenate([W[:_D], W[_D:2 * _D] * scale], axis=0)
    wb = jnp.concatenate([W[2 * _D:3 * _D] * scale, W[3 * _D:] * scale], axis=0)
    b2 = b.reshape(1, _D)

    def mm(ar, br_, war, wbr, biasr, o):
        x = (
            jnp.dot(ar[...], war[...], preferred_element_type=jnp.float32)
            + jnp.dot(br_[...], wbr[...], preferred_element_type=jnp.float32)
            + biasr[...]
        )
        o[...] = x.reshape(_BB, _L, _D)

    return pl.pallas_call(
        mm,
        grid=(_B // _BB,),
        in_specs=[
            pl.BlockSpec((_BB * _L, 128), lambda i: (i, 0)),
            pl.BlockSpec((_BB * _L, 128), lambda i: (i, 0)),
            pl.BlockSpec((128, _D), lambda i: (0, 0)),
            pl.BlockSpec((128, _D), lambda i: (0, 0)),
            pl.BlockSpec((1, _D), lambda i: (0, 0)),
        ],
        out_specs=pl.BlockSpec((_BB, _L, _D), lambda i: (i, 0, 0)),
        out_shape=jax.ShapeDtypeStruct((_B, _L, _D), jnp.float32),
    )(out_a, out_b, wa, wb, b2)


def kernel(item_ids, genre_ids, director_ids, writer_ids, item_table,
           genre_table, director_table, writer_table, W, b):
    nrow = _NW * _NCHUNK
    item_f = item_ids.reshape(nrow, _CHUNK).astype(jnp.int32)
    genre_f = genre_ids.reshape(nrow, _M5).astype(jnp.int32)
    dir_f = director_ids.reshape(nrow, _M5).astype(jnp.int32)
    wri_f = writer_ids.reshape(nrow, _M5).astype(jnp.int32)

    out_a, out_b = _sc_gather_pool(
        item_f, genre_f, dir_f, wri_f,
        item_table, genre_table, director_table, writer_table
    )
    return _tc_project(out_a, out_b, W, b)


# speedup vs baseline: 1.1255x; 1.1252x over previous
import functools

import jax
import jax.numpy as jnp
from jax import lax
from jax.experimental import pallas as pl
from jax.experimental.pallas import tpu as pltpu
from jax.experimental.pallas import tpu_sc as plsc

_B, _L, _D = 4096, 50, 64
_T = _B * _L
_NC, _NS = 2, 16
_NW = _NC * _NS
_TPW = _T // _NW
_CHUNK = 32
_NCHUNK = _TPW // _CHUNK
_R = 16


def _sc_gather_pool(idx_all, item_table, genre_table, director_table, writer_table):
    mesh = plsc.VectorSubcoreMesh(core_axis_name="c", subcore_axis_name="s")

    @functools.partial(
        pl.kernel,
        out_type=(
            jax.ShapeDtypeStruct((_T, 128), jnp.float32),
            jax.ShapeDtypeStruct((_T, 128), jnp.float32),
        ),
        mesh=mesh,
        scratch_types=(
            pltpu.VMEM((2, 4, 128), jnp.int32),
            pltpu.VMEM((2, _CHUNK, _D), jnp.float32),
            pltpu.VMEM((2, 15, _CHUNK, _D), jnp.float32),
            pltpu.VMEM((2, _CHUNK, 128), jnp.float32),
            pltpu.VMEM((2, _CHUNK, 128), jnp.float32),
            pltpu.SemaphoreType.DMA,
            pltpu.SemaphoreType.DMA,
            pltpu.SemaphoreType.DMA,
            pltpu.SemaphoreType.DMA,
        ),
        compiler_params=pltpu.CompilerParams(use_tc_tiling_on_sc=False),
    )
    def k(idx_hbm, item_hbm, genre_hbm, dir_hbm, wri_hbm,
          oa_hbm, ob_hbm, idx_v, item_v, rows_v, oa_v, ob_v,
          gsem0, gsem1, isem0, isem1):
        wid = lax.axis_index("c") * _NS + lax.axis_index("s")
        gsems = (gsem0, gsem1)
        isems = (isem0, isem1)
        tables = (genre_hbm, dir_hbm, wri_hbm)

        def slot_idx(b, r):
            return idx_v.at[b, r // 4, pl.ds((r % 4) * _CHUNK, _CHUNK)]

        def fire(b):
            cps = [pltpu.async_copy(item_hbm.at[slot_idx(b, 0)], item_v.at[b], gsems[b])]
            for t in range(3):
                for m in range(5):
                    r = 1 + 5 * t + m
                    cps.append(pltpu.async_copy(
                        tables[t].at[slot_idx(b, r)], rows_v.at[b, r - 1], gsems[b]))
            return cps

        def drain(b):
            for cp in fire_descs(b):
                cp.wait()

        def fire_descs(b):
            descs = [pltpu.make_async_copy(item_hbm.at[slot_idx(b, 0)], item_v.at[b], gsems[b])]
            for t in range(3):
                for m in range(5):
                    r = 1 + 5 * t + m
                    descs.append(pltpu.make_async_copy(
                        tables[t].at[slot_idx(b, r)], rows_v.at[b, r - 1], gsems[b]))
            return descs

        pltpu.sync_copy(idx_hbm.at[wid, 0], idx_v.at[0])
        fire(0)
        pltpu.async_copy(idx_hbm.at[wid, 1], idx_v.at[1], isems[1])

        @pl.loop(0, _NCHUNK, step=2)
        def _pair(k2):
            for b in range(2):
                kc = k2 + b
                nb = 1 - b

                @pl.when(kc + 1 < _NCHUNK)
                def _():
                    pltpu.make_async_copy(
                        idx_hbm.at[wid, kc + 1], idx_v.at[nb], isems[nb]).wait()
                    fire(nb)

                drain(b)

                @pl.when(kc + 2 < _NCHUNK)
                def _():
                    pltpu.async_copy(idx_hbm.at[wid, kc + 2], idx_v.at[b], isems[b])

                @pl.loop(0, _CHUNK, unroll=4)
                def _tok(i):
                    for c in range(_D // 16):
                        sl = pl.ds(c * 16, 16)
                        oa_v[b, i, sl] = item_v[b, i, sl]
                        for t in range(3):
                            acc = rows_v[b, 5 * t, i, sl]
                            for m in range(1, 5):
                                acc = acc + rows_v[b, 5 * t + m, i, sl]
                            if t == 0:
                                oa_v[b, i, pl.ds(_D + c * 16, 16)] = acc
                            else:
                                ob_v[b, i, pl.ds((t - 1) * _D + c * 16, 16)] = acc

                base = wid * _TPW + kc * _CHUNK
                pltpu.sync_copy(oa_v.at[b], oa_hbm.at[pl.ds(base, _CHUNK)])
                pltpu.sync_copy(ob_v.at[b], ob_hbm.at[pl.ds(base, _CHUNK)])

    return k(idx_all, item_table, genre_table, director_table, writer_table)


_BB = 64


def _tc_project(out_a, out_b, W, b):
    scale = jnp.float32(0.2)
    wa = jnp.concatenate([W[:_D], W[_D:2 * _D] * scale], axis=0)
    wb = jnp.concatenate([W[2 * _D:3 * _D] * scale, W[3 * _D:] * scale], axis=0)
    b2 = b.reshape(1, _D)

    def mm(ar, br_, war, wbr, biasr, o):
        x = (
            jnp.dot(ar[...], war[...], preferred_element_type=jnp.float32)
            + jnp.dot(br_[...], wbr[...], preferred_element_type=jnp.float32)
            + biasr[...]
        )
        o[...] = x.reshape(_BB, _L, _D)

    return pl.pallas_call(
        mm,
        grid=(_B // _BB,),
        in_specs=[
            pl.BlockSpec((_BB * _L, 128), lambda i: (i, 0)),
            pl.BlockSpec((_BB * _L, 128), lambda i: (i, 0)),
            pl.BlockSpec((128, _D), lambda i: (0, 0)),
            pl.BlockSpec((128, _D), lambda i: (0, 0)),
            pl.BlockSpec((1, _D), lambda i: (0, 0)),
        ],
        out_specs=pl.BlockSpec((_BB, _L, _D), lambda i: (i, 0, 0)),
        out_shape=jax.ShapeDtypeStruct((_B, _L, _D), jnp.float32),
    )(out_a, out_b, wa, wb, b2)


def kernel(item_ids, genre_ids, director_ids, writer_ids, item_table,
           genre_table, director_table, writer_table, W, b):
    ii = item_ids.reshape(_T, 1).astype(jnp.int32)
    gi = genre_ids.reshape(_T, 5).astype(jnp.int32)
    di = director_ids.reshape(_T, 5).astype(jnp.int32)
    wi = writer_ids.reshape(_T, 5).astype(jnp.int32)
    stacked = jnp.concatenate([ii, gi, di, wi], axis=1)
    idx_all = (
        stacked.reshape(_NW, _NCHUNK, _CHUNK, _R)
        .transpose(0, 1, 3, 2)
        .reshape(_NW, _NCHUNK, 4, 128)
    )

    out_a, out_b = _sc_gather_pool(
        idx_all, item_table, genre_table, director_table, writer_table
    )
    return _tc_project(out_a, out_b, W, b)
